# baseline (speedup 1.0000x reference)
import dataclasses
import functools

import jax
import jax.numpy as jnp
from jax import lax
from jax.experimental import pallas as pl
from jax.experimental.pallas import tpu as pltpu
from jax.experimental.pallas import tpu_sc as plsc

_NC = 2
_NS = 16
_NW = _NC * _NS
_LANES = 16
_CHUNK = 128
_NEG_SLOPE = 0.01


def _sc_mesh():
    return plsc.VectorSubcoreMesh(core_axis_name="c", subcore_axis_name="s")


def _sc_compiler_params():
    cp = pltpu.CompilerParams()
    if "needs_layout_passes" in pltpu.CompilerParams.__dataclass_fields__:
        cp = dataclasses.replace(cp, needs_layout_passes=False)
    return cp


def _sc_degree(idxm, np_, kw):
    @functools.partial(
        pl.kernel,
        out_type=jax.ShapeDtypeStruct((_NW, np_), jnp.float32),
        mesh=_sc_mesh(),
        compiler_params=_sc_compiler_params(),
        scratch_types=[
            pltpu.VMEM((1, _CHUNK), jnp.int32),
            pltpu.VMEM((np_,), jnp.float32),
        ],
    )
    def k(idxm_hbm, out_hbm, dstv, deg_local):
        c = lax.axis_index("c")
        s = lax.axis_index("s")
        w = s * _NC + c
        zero = jnp.zeros((_LANES,), jnp.float32)

        @pl.loop(0, np_, step=_LANES)
        def _(i):
            deg_local[pl.ds(i, _LANES)] = zero

        ones = jnp.ones((_LANES,), jnp.float32)

        @pl.loop(0, kw)
        def _(j):
            pltpu.sync_copy(idxm_hbm.at[(w * kw + j) * 2 + 1], dstv.at[0])

            @pl.loop(0, _CHUNK, step=_LANES)
            def _(t):
                idx = dstv[0, pl.ds(t, _LANES)]
                plsc.addupdate_scatter(deg_local, [idx], ones)

        pltpu.sync_copy(deg_local, out_hbm.at[w])

    return k(idxm)


def _sc_aggregate(ht, idxm, np_, kw):
    rt = np_ // _NS

    @functools.partial(
        pl.kernel,
        out_type=(jax.ShapeDtypeStruct((np_, 128), jnp.float32),
                  jax.ShapeDtypeStruct((np_, 128), jnp.float32)),
        mesh=_sc_mesh(),
        scratch_types=[
            pltpu.VMEM((2 * kw, _CHUNK), jnp.int32),
            pltpu.VMEM((_CHUNK, 128), jnp.float32),
            pltpu.VMEM_SHARED((np_, 128), jnp.float32),
        ],
    )
    def k(ht_hbm, idxm_hbm, out0_hbm, out1_hbm, idx_v, rows, acc):
        c = lax.axis_index("c")
        s = lax.axis_index("s")
        w = s * _NC + c
        zero = jnp.zeros((_LANES,), jnp.float32)

        @pl.loop(0, _CHUNK)
        def _(r):
            @pl.loop(0, 128, step=_LANES)
            def _(t):
                rows[r, pl.ds(t, _LANES)] = zero

        @pl.loop(0, rt, step=_CHUNK)
        def _(i):
            pltpu.sync_copy(rows, acc.at[pl.ds(s * rt + i, _CHUNK)])

        pltpu.sync_copy(idxm_hbm.at[pl.ds(w * kw * 2, 2 * kw)], idx_v)
        plsc.subcore_barrier()

        @pl.loop(0, kw)
        def _(j):
            pltpu.sync_copy(ht_hbm.at[idx_v.at[2 * j]], rows)
            pltpu.sync_copy(rows, acc.at[idx_v.at[2 * j + 1]], add=True)

        plsc.subcore_barrier()

        @pl.when(c == 0)
        def _():
            pltpu.sync_copy(acc.at[pl.ds(s * rt, rt)], out0_hbm.at[pl.ds(s * rt, rt)])

        @pl.when(c == 1)
        def _():
            pltpu.sync_copy(acc.at[pl.ds(s * rt, rt)], out1_hbm.at[pl.ds(s * rt, rt)])

    return k(ht, idxm)


def _tc_dis(degp, np_):
    def body(deg_ref, out_ref):
        ssum = jnp.sum(deg_ref[...], axis=0, keepdims=True)
        r = lax.rsqrt(1.0 + ssum)
        rows = jnp.broadcast_to(r, (128, 128))
        eye = jnp.eye(128, dtype=jnp.float32)
        colmat = lax.dot_general(
            rows, eye, (((0,), (0,)), ((), ())),
            preferred_element_type=jnp.float32)
        out_ref[...] = colmat

    return pl.pallas_call(
        body,
        grid=(np_ // 128,),
        in_specs=[pl.BlockSpec((_NW, 128), lambda i: (0, i))],
        out_specs=pl.BlockSpec((128, 128), lambda i: (i, 0)),
        out_shape=jax.ShapeDtypeStruct((np_, 128), jnp.float32),
    )(degp)


_RB = 512


def _tc_matmul_scale(xp, W, disf, np_):
    def body(x_ref, w_ref, d_ref, o_ref):
        h = jnp.dot(x_ref[...], w_ref[...], preferred_element_type=jnp.float32)
        o_ref[...] = h * d_ref[...]

    return pl.pallas_call(
        body,
        grid=(np_ // _RB,),
        in_specs=[
            pl.BlockSpec((_RB, 128), lambda i: (i, 0)),
            pl.BlockSpec((128, 128), lambda i: (0, 0)),
            pl.BlockSpec((_RB, 128), lambda i: (i, 0)),
        ],
        out_specs=pl.BlockSpec((_RB, 128), lambda i: (i, 0)),
        out_shape=jax.ShapeDtypeStruct((np_, 128), jnp.float32),
    )(xp, W, disf)


def _tc_layer(p0, p1, ht, disf, b2, W, np_):
    def body(p0_ref, p1_ref, h_ref, d_ref, b_ref, w_ref, o_ref):
        y = d_ref[...] * (p0_ref[...] + p1_ref[...] + h_ref[...]) + b_ref[...]
        y = jnp.where(y >= 0, y, _NEG_SLOPE * y)
        h2 = jnp.dot(y, w_ref[...], preferred_element_type=jnp.float32)
        o_ref[...] = h2 * d_ref[...]

    row = pl.BlockSpec((_RB, 128), lambda i: (i, 0))
    return pl.pallas_call(
        body,
        grid=(np_ // _RB,),
        in_specs=[
            row, row, row, row,
            pl.BlockSpec((1, 128), lambda i: (0, 0)),
            pl.BlockSpec((128, 128), lambda i: (0, 0)),
        ],
        out_specs=row,
        out_shape=jax.ShapeDtypeStruct((np_, 128), jnp.float32),
    )(p0, p1, ht, disf, b2, W)


def _tc_final(p0, p1, ht, disf, b2, np_):
    def body(p0_ref, p1_ref, h_ref, d_ref, b_ref, o_ref):
        y = d_ref[...] * (p0_ref[...] + p1_ref[...] + h_ref[...]) + b_ref[...]
        o_ref[...] = jnp.where(y >= 0, y, _NEG_SLOPE * y)

    row = pl.BlockSpec((_RB, 128), lambda i: (i, 0))
    return pl.pallas_call(
        body,
        grid=(np_ // _RB,),
        in_specs=[row, row, row, row, pl.BlockSpec((1, 128), lambda i: (0, 0))],
        out_specs=row,
        out_shape=jax.ShapeDtypeStruct((np_, 128), jnp.float32),
    )(p0, p1, ht, disf, b2)


def kernel(x, edge_index, W1, b1, W2, b2, W3, b3, W4, b4):
    n, _ = x.shape
    e = edge_index.shape[1]
    np_ = -(-n // (_NS * _CHUNK)) * (_NS * _CHUNK)
    ew = -(-e // (_NW * 2 * _CHUNK)) * 2 * _CHUNK
    kw = ew // _CHUNK
    ep = ew * _NW

    src = edge_index[0]
    dst = edge_index[1]
    pad = ep - e
    junk = n + jnp.arange(pad, dtype=edge_index.dtype) % (np_ - n)
    srcp = jnp.concatenate([src, jnp.zeros((pad,), edge_index.dtype)])
    dstp = jnp.concatenate([dst, junk])
    idxm = jnp.stack(
        [srcp.reshape(-1, _CHUNK), dstp.reshape(-1, _CHUNK)], axis=1
    ).reshape(-1, _CHUNK)
    xp = jnp.pad(x, ((0, np_ - n), (0, 0)))

    degp = _sc_degree(idxm, np_, kw)
    disf = _tc_dis(degp, np_)

    ht = _tc_matmul_scale(xp, W1, disf, np_)
    for b, w_next in ((b1, W2), (b2, W3), (b3, W4)):
        p0, p1 = _sc_aggregate(ht, idxm, np_, kw)
        ht = _tc_layer(p0, p1, ht, disf, b.reshape(1, -1), w_next, np_)
    p0, p1 = _sc_aggregate(ht, idxm, np_, kw)
    out = _tc_final(p0, p1, ht, disf, b4.reshape(1, -1), np_)
    return out[:n]


# speedup vs baseline: 1.4109x; 1.4109x over previous
import dataclasses
import functools

import jax
import jax.numpy as jnp
from jax import lax
from jax.experimental import pallas as pl
from jax.experimental.pallas import tpu as pltpu
from jax.experimental.pallas import tpu_sc as plsc

_NC = 2
_NS = 16
_NW = _NC * _NS
_LANES = 16
_CHUNK = 128
_NEG_SLOPE = 0.01


def _sc_mesh():
    return plsc.VectorSubcoreMesh(core_axis_name="c", subcore_axis_name="s")


def _sc_compiler_params():
    cp = pltpu.CompilerParams()
    if "needs_layout_passes" in pltpu.CompilerParams.__dataclass_fields__:
        cp = dataclasses.replace(cp, needs_layout_passes=False)
    return cp


def _sc_degree(idxm, np_, kw):
    @functools.partial(
        pl.kernel,
        out_type=jax.ShapeDtypeStruct((_NW, np_), jnp.float32),
        mesh=_sc_mesh(),
        compiler_params=_sc_compiler_params(),
        scratch_types=[
            pltpu.VMEM((1, _CHUNK), jnp.int32),
            pltpu.VMEM((np_,), jnp.float32),
        ],
    )
    def k(idxm_hbm, out_hbm, dstv, deg_local):
        c = lax.axis_index("c")
        s = lax.axis_index("s")
        w = s * _NC + c
        zero = jnp.zeros((_LANES,), jnp.float32)

        @pl.loop(0, np_, step=_LANES)
        def _(i):
            deg_local[pl.ds(i, _LANES)] = zero

        ones = jnp.ones((_LANES,), jnp.float32)

        @pl.loop(0, kw)
        def _(j):
            pltpu.sync_copy(idxm_hbm.at[(w * kw + j) * 2 + 1], dstv.at[0])

            @pl.loop(0, _CHUNK, step=_LANES)
            def _(t):
                idx = dstv[0, pl.ds(t, _LANES)]
                plsc.addupdate_scatter(deg_local, [idx], ones)

        pltpu.sync_copy(deg_local, out_hbm.at[w])

    return k(idxm)


def _sc_aggregate(ht, idxm, np_, kw):
    rt = np_ // _NS

    @functools.partial(
        pl.kernel,
        out_type=(jax.ShapeDtypeStruct((np_, 128), jnp.float32),
                  jax.ShapeDtypeStruct((np_, 128), jnp.float32)),
        mesh=_sc_mesh(),
        scratch_types=[
            pltpu.VMEM((2, _CHUNK), jnp.int32),
            pltpu.VMEM((_CHUNK, 128), jnp.float32),
            pltpu.VMEM_SHARED((np_, 128), jnp.float32),
        ],
    )
    def k(ht_hbm, idxm_hbm, out0_hbm, out1_hbm, idx_v, rows, acc):
        c = lax.axis_index("c")
        s = lax.axis_index("s")
        w = s * _NC + c
        zero = jnp.zeros((_LANES,), jnp.float32)

        @pl.loop(0, _CHUNK)
        def _(r):
            @pl.loop(0, 128, step=_LANES)
            def _(t):
                rows[r, pl.ds(t, _LANES)] = zero

        @pl.loop(0, rt, step=_CHUNK)
        def _(i):
            pltpu.sync_copy(rows, acc.at[pl.ds(s * rt + i, _CHUNK)])

        plsc.subcore_barrier()

        @pl.loop(0, kw)
        def _(j):
            pltpu.sync_copy(idxm_hbm.at[pl.ds((w * kw + j) * 2, 2)], idx_v)
            pltpu.sync_copy(ht_hbm.at[idx_v.at[0]], rows)
            pltpu.sync_copy(rows, acc.at[idx_v.at[1]], add=True)

        plsc.subcore_barrier()

        @pl.when(c == 0)
        def _():
            pltpu.sync_copy(acc.at[pl.ds(s * rt, rt)], out0_hbm.at[pl.ds(s * rt, rt)])

        @pl.when(c == 1)
        def _():
            pltpu.sync_copy(acc.at[pl.ds(s * rt, rt)], out1_hbm.at[pl.ds(s * rt, rt)])

    return k(ht, idxm)


def _tc_dis(degp, np_):
    def body(deg_ref, out_ref):
        ssum = jnp.sum(deg_ref[...], axis=0, keepdims=True)
        r = lax.rsqrt(1.0 + ssum)
        rows = jnp.broadcast_to(r, (128, 128))
        eye = jnp.eye(128, dtype=jnp.float32)
        colmat = lax.dot_general(
            rows, eye, (((0,), (0,)), ((), ())),
            preferred_element_type=jnp.float32)
        out_ref[...] = colmat

    return pl.pallas_call(
        body,
        grid=(np_ // 128,),
        in_specs=[pl.BlockSpec((_NW, 128), lambda i: (0, i))],
        out_specs=pl.BlockSpec((128, 128), lambda i: (i, 0)),
        out_shape=jax.ShapeDtypeStruct((np_, 128), jnp.float32),
    )(degp)


_RB = 512


def _tc_matmul_scale(xp, W, disf, np_):
    def body(x_ref, w_ref, d_ref, o_ref):
        h = jnp.dot(x_ref[...], w_ref[...], preferred_element_type=jnp.float32)
        o_ref[...] = h * d_ref[...]

    return pl.pallas_call(
        body,
        grid=(np_ // _RB,),
        in_specs=[
            pl.BlockSpec((_RB, 128), lambda i: (i, 0)),
            pl.BlockSpec((128, 128), lambda i: (0, 0)),
            pl.BlockSpec((_RB, 128), lambda i: (i, 0)),
        ],
        out_specs=pl.BlockSpec((_RB, 128), lambda i: (i, 0)),
        out_shape=jax.ShapeDtypeStruct((np_, 128), jnp.float32),
    )(xp, W, disf)


def _tc_layer(p0, p1, ht, disf, b2, W, np_):
    def body(p0_ref, p1_ref, h_ref, d_ref, b_ref, w_ref, o_ref):
        y = d_ref[...] * (p0_ref[...] + p1_ref[...] + h_ref[...]) + b_ref[...]
        y = jnp.where(y >= 0, y, _NEG_SLOPE * y)
        h2 = jnp.dot(y, w_ref[...], preferred_element_type=jnp.float32)
        o_ref[...] = h2 * d_ref[...]

    row = pl.BlockSpec((_RB, 128), lambda i: (i, 0))
    return pl.pallas_call(
        body,
        grid=(np_ // _RB,),
        in_specs=[
            row, row, row, row,
            pl.BlockSpec((1, 128), lambda i: (0, 0)),
            pl.BlockSpec((128, 128), lambda i: (0, 0)),
        ],
        out_specs=row,
        out_shape=jax.ShapeDtypeStruct((np_, 128), jnp.float32),
    )(p0, p1, ht, disf, b2, W)


def _tc_final(p0, p1, ht, disf, b2, np_):
    def body(p0_ref, p1_ref, h_ref, d_ref, b_ref, o_ref):
        y = d_ref[...] * (p0_ref[...] + p1_ref[...] + h_ref[...]) + b_ref[...]
        o_ref[...] = jnp.where(y >= 0, y, _NEG_SLOPE * y)

    row = pl.BlockSpec((_RB, 128), lambda i: (i, 0))
    return pl.pallas_call(
        body,
        grid=(np_ // _RB,),
        in_specs=[row, row, row, row, pl.BlockSpec((1, 128), lambda i: (0, 0))],
        out_specs=row,
        out_shape=jax.ShapeDtypeStruct((np_, 128), jnp.float32),
    )(p0, p1, ht, disf, b2)


def kernel(x, edge_index, W1, b1, W2, b2, W3, b3, W4, b4):
    n, _ = x.shape
    e = edge_index.shape[1]
    np_ = -(-n // (_NS * _CHUNK)) * (_NS * _CHUNK)
    ew = -(-e // (_NW * _CHUNK)) * _CHUNK
    kw = ew // _CHUNK
    ep = ew * _NW

    src = edge_index[0]
    dst = edge_index[1]
    pad = ep - e
    srcp = jnp.concatenate([src, jnp.zeros((pad,), edge_index.dtype)])
    dstp = jnp.concatenate(
        [dst, jnp.full((pad,), np_ - 1, edge_index.dtype)])
    idxm = jnp.stack(
        [srcp.reshape(-1, _CHUNK), dstp.reshape(-1, _CHUNK)], axis=1
    ).reshape(-1, _CHUNK)
    xp = jnp.pad(x, ((0, np_ - n), (0, 0)))

    degp = _sc_degree(idxm, np_, kw)
    disf = _tc_dis(degp, np_)

    ht = _tc_matmul_scale(xp, W1, disf, np_)
    for b, w_next in ((b1, W2), (b2, W3), (b3, W4)):
        p0, p1 = _sc_aggregate(ht, idxm, np_, kw)
        ht = _tc_layer(p0, p1, ht, disf, b.reshape(1, -1), w_next, np_)
    p0, p1 = _sc_aggregate(ht, idxm, np_, kw)
    out = _tc_final(p0, p1, ht, disf, b4.reshape(1, -1), np_)
    return out[:n]


# speedup vs baseline: 1.4116x; 1.0005x over previous
import dataclasses
import functools

import jax
import jax.numpy as jnp
from jax import lax
from jax.experimental import pallas as pl
from jax.experimental.pallas import tpu as pltpu
from jax.experimental.pallas import tpu_sc as plsc

_NC = 2
_NS = 16
_NW = _NC * _NS
_LANES = 16
_CHUNK = 128
_NEG_SLOPE = 0.01


def _sc_mesh():
    return plsc.VectorSubcoreMesh(core_axis_name="c", subcore_axis_name="s")


def _sc_compiler_params():
    cp = pltpu.CompilerParams()
    if "needs_layout_passes" in pltpu.CompilerParams.__dataclass_fields__:
        cp = dataclasses.replace(cp, needs_layout_passes=False)
    return cp


def _sc_degree(idxm, np_, kw):
    @functools.partial(
        pl.kernel,
        out_type=jax.ShapeDtypeStruct((_NW, np_), jnp.float32),
        mesh=_sc_mesh(),
        compiler_params=_sc_compiler_params(),
        scratch_types=[
            pltpu.VMEM((1, _CHUNK), jnp.int32),
            pltpu.VMEM((np_,), jnp.float32),
        ],
    )
    def k(idxm_hbm, out_hbm, dstv, deg_local):
        c = lax.axis_index("c")
        s = lax.axis_index("s")
        w = s * _NC + c
        zero = jnp.zeros((_LANES,), jnp.float32)

        @pl.loop(0, np_, step=_LANES)
        def _(i):
            deg_local[pl.ds(i, _LANES)] = zero

        ones = jnp.ones((_LANES,), jnp.float32)

        @pl.loop(0, kw)
        def _(j):
            pltpu.sync_copy(idxm_hbm.at[(w * kw + j) * 2 + 1], dstv.at[0])

            @pl.loop(0, _CHUNK, step=_LANES)
            def _(t):
                idx = dstv[0, pl.ds(t, _LANES)]
                plsc.addupdate_scatter(deg_local, [idx], ones)

        pltpu.sync_copy(deg_local, out_hbm.at[w])

    return k(idxm)


def _sc_aggregate(ht, idxm, np_, kw):
    rt = np_ // _NS

    @functools.partial(
        pl.kernel,
        out_type=(jax.ShapeDtypeStruct((np_, 128), jnp.float32),
                  jax.ShapeDtypeStruct((np_, 128), jnp.float32)),
        mesh=_sc_mesh(),
        scratch_types=[
            pltpu.VMEM((2, _CHUNK), jnp.int32),
            pltpu.VMEM((_CHUNK, 128), jnp.float32),
            pltpu.VMEM_SHARED((np_, 128), jnp.float32),
        ],
    )
    def k(ht_hbm, idxm_hbm, out0_hbm, out1_hbm, idx_v, rows, acc):
        c = lax.axis_index("c")
        s = lax.axis_index("s")
        w = s * _NC + c
        zero = jnp.zeros((_LANES,), jnp.float32)

        @pl.loop(0, _CHUNK)
        def _(r):
            @pl.loop(0, 128, step=_LANES)
            def _(t):
                rows[r, pl.ds(t, _LANES)] = zero

        @pl.loop(0, rt, step=_CHUNK)
        def _(i):
            pltpu.sync_copy(rows, acc.at[pl.ds(s * rt + i, _CHUNK)])

        plsc.subcore_barrier()

        @pl.loop(0, kw)
        def _(j):
            pltpu.sync_copy(idxm_hbm.at[pl.ds((w * kw + j) * 2, 2)], idx_v)
            pltpu.sync_copy(ht_hbm.at[idx_v.at[0]], rows)
            pltpu.sync_copy(rows, acc.at[idx_v.at[1]], add=True)

        plsc.subcore_barrier()

        @pl.when(c == 0)
        def _():
            pltpu.sync_copy(acc.at[pl.ds(s * rt, rt)], out0_hbm.at[pl.ds(s * rt, rt)])

        @pl.when(c == 1)
        def _():
            pltpu.sync_copy(acc.at[pl.ds(s * rt, rt)], out1_hbm.at[pl.ds(s * rt, rt)])

    return k(ht, idxm)


def _tc_dis(degp, np_):
    def body(deg_ref, out_ref):
        ssum = jnp.sum(deg_ref[...], axis=0, keepdims=True)
        r = lax.rsqrt(1.0 + ssum)
        rows = jnp.broadcast_to(r, (128, 128))
        eye = jnp.eye(128, dtype=jnp.float32)
        colmat = lax.dot_general(
            rows, eye, (((0,), (0,)), ((), ())),
            preferred_element_type=jnp.float32)
        out_ref[...] = colmat

    return pl.pallas_call(
        body,
        grid=(np_ // 128,),
        in_specs=[pl.BlockSpec((_NW, 128), lambda i: (0, i))],
        out_specs=pl.BlockSpec((128, 128), lambda i: (i, 0)),
        out_shape=jax.ShapeDtypeStruct((np_, 128), jnp.float32),
    )(degp)


_RB = 512


def _tc_matmul_scale(xp, W, disf, np_):
    def body(x_ref, w_ref, d_ref, o_ref):
        h = jnp.dot(x_ref[...], w_ref[...], preferred_element_type=jnp.float32)
        o_ref[...] = h * d_ref[...]

    return pl.pallas_call(
        body,
        grid=(np_ // _RB,),
        in_specs=[
            pl.BlockSpec((_RB, 128), lambda i: (i, 0)),
            pl.BlockSpec((128, 128), lambda i: (0, 0)),
            pl.BlockSpec((_RB, 128), lambda i: (i, 0)),
        ],
        out_specs=pl.BlockSpec((_RB, 128), lambda i: (i, 0)),
        out_shape=jax.ShapeDtypeStruct((np_, 128), jnp.float32),
    )(xp, W, disf)


def _tc_layer(p0, p1, ht, disf, b2, W, np_):
    def body(p0_ref, p1_ref, h_ref, d_ref, b_ref, w_ref, o_ref):
        y = d_ref[...] * (p0_ref[...] + p1_ref[...] + h_ref[...]) + b_ref[...]
        y = jnp.where(y >= 0, y, _NEG_SLOPE * y)
        h2 = jnp.dot(y, w_ref[...], preferred_element_type=jnp.float32)
        o_ref[...] = h2 * d_ref[...]

    row = pl.BlockSpec((_RB, 128), lambda i: (i, 0))
    return pl.pallas_call(
        body,
        grid=(np_ // _RB,),
        in_specs=[
            row, row, row, row,
            pl.BlockSpec((1, 128), lambda i: (0, 0)),
            pl.BlockSpec((128, 128), lambda i: (0, 0)),
        ],
        out_specs=row,
        out_shape=jax.ShapeDtypeStruct((np_, 128), jnp.float32),
    )(p0, p1, ht, disf, b2, W)


def _tc_final(p0, p1, ht, disf, b2, np_):
    def body(p0_ref, p1_ref, h_ref, d_ref, b_ref, o_ref):
        y = d_ref[...] * (p0_ref[...] + p1_ref[...] + h_ref[...]) + b_ref[...]
        o_ref[...] = jnp.where(y >= 0, y, _NEG_SLOPE * y)

    row = pl.BlockSpec((_RB, 128), lambda i: (i, 0))
    return pl.pallas_call(
        body,
        grid=(np_ // _RB,),
        in_specs=[row, row, row, row, pl.BlockSpec((1, 128), lambda i: (0, 0))],
        out_specs=row,
        out_shape=jax.ShapeDtypeStruct((np_, 128), jnp.float32),
    )(p0, p1, ht, disf, b2)


def kernel(x, edge_index, W1, b1, W2, b2, W3, b3, W4, b4):
    n, _ = x.shape
    e = edge_index.shape[1]
    np_ = -(-n // (_NS * _CHUNK)) * (_NS * _CHUNK)
    ew = -(-e // (_NW * _CHUNK)) * _CHUNK
    kw = ew // _CHUNK
    ep = ew * _NW

    src = edge_index[0]
    dst = edge_index[1]
    pad = ep - e
    junk = n + jnp.arange(pad, dtype=edge_index.dtype) % (np_ - n)
    srcp = jnp.concatenate([src, jnp.zeros((pad,), edge_index.dtype)])
    dstp = jnp.concatenate([dst, junk])
    idxm = jnp.stack(
        [srcp.reshape(-1, _CHUNK), dstp.reshape(-1, _CHUNK)], axis=1
    ).reshape(-1, _CHUNK)
    xp = jnp.pad(x, ((0, np_ - n), (0, 0)))

    degp = _sc_degree(idxm, np_, kw)
    disf = _tc_dis(degp, np_)

    ht = _tc_matmul_scale(xp, W1, disf, np_)
    for b, w_next in ((b1, W2), (b2, W3), (b3, W4)):
        p0, p1 = _sc_aggregate(ht, idxm, np_, kw)
        ht = _tc_layer(p0, p1, ht, disf, b.reshape(1, -1), w_next, np_)
    p0, p1 = _sc_aggregate(ht, idxm, np_, kw)
    out = _tc_final(p0, p1, ht, disf, b4.reshape(1, -1), np_)
    return out[:n]
